# baseline (speedup 1.0000x reference)
import functools

import jax
import jax.numpy as jnp
from jax import lax
from jax.experimental import pallas as pl
from jax.experimental.pallas import tpu as pltpu
from jax.experimental.pallas import tpu_sc as plsc

N = 10000
E = 320000
ELEM_EMB = 92
NBR = 16
FEA = 128

EH = E // 2
EH_PAD = -(-EH // 4096) * 4096
EB = 1280
NB = 1000
EPS = 1e-5

_NC = 2
_NS = 16
GW = 128
DOFF = 624
DSZ = 640


@functools.cache
def _sc_mesh():
    return plsc.VectorSubcoreMesh(core_axis_name="core", subcore_axis_name="subcore")


def _embed_body(a_ref, w_ref, b_ref, o_ref):
    o_ref[...] = (
        jnp.dot(a_ref[...], w_ref[...], preferred_element_type=jnp.float32)
        + b_ref[...]
    )


def tc_embed(atom_pad, w_pad, b):
    return pl.pallas_call(
        _embed_body,
        grid=(N // NB,),
        in_specs=[
            pl.BlockSpec((NB, 128), lambda i: (i, 0)),
            pl.BlockSpec((128, FEA), lambda i: (0, 0)),
            pl.BlockSpec((1, FEA), lambda i: (0, 0)),
        ],
        out_specs=pl.BlockSpec((NB, FEA), lambda i: (i, 0)),
        out_shape=jax.ShapeDtypeStruct((N, FEA), jnp.float32),
    )(atom_pad, w_pad, b)


def _edge_linear(xs_ref, xn_ref, nb_ref, w_ref, b_ref):
    xs = xs_ref[...].astype(jnp.bfloat16)
    xn = xn_ref[...].astype(jnp.bfloat16)
    nb = nb_ref[...]
    t = jnp.dot(xs, w_ref[0:FEA, :], preferred_element_type=jnp.float32)
    t += jnp.dot(xn, w_ref[FEA:2 * FEA, :], preferred_element_type=jnp.float32)
    t += jnp.dot(nb, w_ref[2 * FEA:2 * FEA + NBR, :], preferred_element_type=jnp.float32)
    return t + b_ref[...]


def _stats_body(xs_ref, xn_ref, nb_ref, w_ref, b_ref, o_ref):
    i = pl.program_id(0)
    t = _edge_linear(xs_ref, xn_ref, nb_ref, w_ref, b_ref)

    @pl.when(i == 0)
    def _():
        o_ref[...] = jnp.zeros_like(o_ref)

    o_ref[0:1, :] += jnp.sum(t, axis=0, keepdims=True)
    o_ref[1:2, :] += jnp.sum(t * t, axis=0, keepdims=True)


def tc_stats(G, nbr_fea, W, b2d):
    neb = G.shape[0] // 2 // EB
    return pl.pallas_call(
        _stats_body,
        grid=(neb,),
        in_specs=[
            pl.BlockSpec((EB, FEA), lambda i: (i, 0)),
            pl.BlockSpec((EB, FEA), lambda i: (neb + i, 0)),
            pl.BlockSpec((EB, NBR), lambda i: (i, 0)),
            pl.BlockSpec((2 * FEA + NBR, 2 * FEA), lambda i: (0, 0)),
            pl.BlockSpec((1, 2 * FEA), lambda i: (0, 0)),
        ],
        out_specs=pl.BlockSpec((8, 2 * FEA), lambda i: (0, 0)),
        out_shape=jax.ShapeDtypeStruct((8, 2 * FEA), jnp.float32),
    )(G, G, nbr_fea, W, b2d)


def _msg_body(xs_ref, xn_ref, nb_ref, w_ref, b_ref, g1_ref, be1_ref,
              sta_ref, stb_ref, o_ref, *, neb_real):
    i = pl.program_id(0)

    @pl.when(i < neb_real)
    def _():
        t = _edge_linear(xs_ref, xn_ref, nb_ref, w_ref, b_ref)
        ssum = sta_ref[0:1, :] + stb_ref[0:1, :]
        ssq = sta_ref[1:2, :] + stb_ref[1:2, :]
        mean = ssum * (1.0 / E)
        var = ssq * (1.0 / E) - mean * mean
        scale = g1_ref[...] * lax.rsqrt(var + EPS)
        y = (t - mean) * scale + be1_ref[...]
        f = y[:, :FEA]
        c = y[:, FEA:]
        sig = 1.0 / (1.0 + jnp.exp(-f))
        sp = jnp.maximum(c, 0.0) + jnp.log1p(jnp.exp(-jnp.abs(c)))
        o_ref[...] = sig * sp

    @pl.when(i >= neb_real)
    def _():
        o_ref[...] = jnp.zeros_like(o_ref)


def tc_msg(G, nbr_fea, W, b2d, g1, be1, sta, stb):
    neb = G.shape[0] // 2 // EB
    nebp = EH_PAD // EB
    clamp = lambda i: jnp.minimum(i, neb - 1)
    return pl.pallas_call(
        functools.partial(_msg_body, neb_real=neb),
        grid=(nebp,),
        in_specs=[
            pl.BlockSpec((EB, FEA), lambda i: (clamp(i), 0)),
            pl.BlockSpec((EB, FEA), lambda i: (neb + clamp(i), 0)),
            pl.BlockSpec((EB, NBR), lambda i: (clamp(i), 0)),
            pl.BlockSpec((2 * FEA + NBR, 2 * FEA), lambda i: (0, 0)),
            pl.BlockSpec((1, 2 * FEA), lambda i: (0, 0)),
            pl.BlockSpec((1, 2 * FEA), lambda i: (0, 0)),
            pl.BlockSpec((1, 2 * FEA), lambda i: (0, 0)),
            pl.BlockSpec((8, 2 * FEA), lambda i: (0, 0)),
            pl.BlockSpec((8, 2 * FEA), lambda i: (0, 0)),
        ],
        out_specs=pl.BlockSpec((EB, FEA), lambda i: (i, 0)),
        out_shape=jax.ShapeDtypeStruct((EH_PAD, FEA), jnp.float32),
    )(G, G, nbr_fea, W, b2d, g1, be1, sta, stb)


def _fapply_body(x_ref, pa0_ref, pa1_ref, pb0_ref, pb1_ref, g2_ref, be2_ref,
                 o_ref, st_ref):
    ph = pl.program_id(0)
    i = pl.program_id(1)
    s = (pa0_ref[0] + pa1_ref[0]) + (pb0_ref[0] + pb1_ref[0])

    @pl.when(ph == 0)
    def _():
        @pl.when(i == 0)
        def _():
            st_ref[...] = jnp.zeros_like(st_ref)

        st_ref[0:1, :] += jnp.sum(s, axis=0, keepdims=True)
        st_ref[1:2, :] += jnp.sum(s * s, axis=0, keepdims=True)

    @pl.when(ph == 1)
    def _():
        mean = st_ref[0:1, :] * (1.0 / N)
        var = st_ref[1:2, :] * (1.0 / N) - mean * mean
        scale = g2_ref[...] * lax.rsqrt(var + EPS)
        y = (s - mean) * scale + be2_ref[...]
        z = x_ref[...] + y
        o_ref[...] = jnp.maximum(z, 0.0) + jnp.log1p(jnp.exp(-jnp.abs(z)))


def tc_fapply(x, parts_a, parts_b, g2, be2):
    return pl.pallas_call(
        _fapply_body,
        grid=(2, N // NB),
        in_specs=[
            pl.BlockSpec((NB, FEA), lambda p, i: (i, 0)),
            pl.BlockSpec((1, NB, FEA), lambda p, i: (0, i, 0)),
            pl.BlockSpec((1, NB, FEA), lambda p, i: (1, i, 0)),
            pl.BlockSpec((1, NB, FEA), lambda p, i: (0, i, 0)),
            pl.BlockSpec((1, NB, FEA), lambda p, i: (1, i, 0)),
            pl.BlockSpec((1, FEA), lambda p, i: (0, 0)),
            pl.BlockSpec((1, FEA), lambda p, i: (0, 0)),
        ],
        out_specs=pl.BlockSpec((NB, FEA), lambda p, i: (i, 0)),
        out_shape=jax.ShapeDtypeStruct((N, FEA), jnp.float32),
        scratch_shapes=[pltpu.VMEM((8, FEA), jnp.float32)],
    )(x, parts_a, parts_a, parts_b, parts_b, g2, be2)


def sc_gather(x, idx2d):
    ng = idx2d.shape[1]

    @functools.partial(
        pl.kernel,
        out_type=jax.ShapeDtypeStruct((ng, FEA), jnp.float32),
        mesh=_sc_mesh(),
        scratch_types=[pltpu.VMEM_SHARED((N, FEA), jnp.float32)],
    )
    def kern(x_hbm, i_hbm, o_hbm, x_sh):
        s = lax.axis_index("subcore")
        pltpu.sync_copy(x_hbm.at[pl.ds(s * DOFF, DSZ)], x_sh.at[pl.ds(s * DOFF, DSZ)])
        plsc.subcore_barrier()

        def body(i_vmem, o_vmem):
            pltpu.sync_copy(x_sh.at[i_vmem.at[0]], o_vmem)

        pltpu.emit_pipeline(
            body,
            grid=(ng // GW,),
            in_specs=[pl.BlockSpec((1, GW), lambda i: (0, i))],
            out_specs=[pl.BlockSpec((GW, FEA), lambda i: (i, 0))],
            core_axis_name=("core", "subcore"),
            dimension_semantics=(pltpu.PARALLEL,),
        )(i_hbm, o_hbm)

    return kern(x, idx2d)


def sc_scatter(msg, si, zeros):
    e_cnt = msg.shape[0]
    epc = e_cnt // _NC
    epsub = epc // _NS
    sw = 128
    nwin = epsub // sw

    @functools.partial(
        pl.kernel,
        out_type=jax.ShapeDtypeStruct((_NC, N, FEA), jnp.float32),
        mesh=_sc_mesh(),
        scratch_types=[
            pltpu.VMEM((sw, FEA), jnp.float32),
            pltpu.VMEM((sw, FEA), jnp.float32),
            pltpu.VMEM((sw,), jnp.int32),
            pltpu.VMEM((sw,), jnp.int32),
            pltpu.VMEM_SHARED((N, FEA), jnp.float32),
            pltpu.SemaphoreType.DMA,
            pltpu.SemaphoreType.DMA,
            pltpu.SemaphoreType.DMA,
            pltpu.SemaphoreType.DMA,
            pltpu.SemaphoreType.DMA,
            pltpu.SemaphoreType.DMA,
        ],
    )
    def kern(msg_hbm, si_hbm, z_hbm, o_hbm, rows0, rows1, idx0, idx1, acc_sh,
             sem_i0, sem_r0, sem_i1, sem_r1, sem_a0, sem_a1):
        c = lax.axis_index("core")
        s = lax.axis_index("subcore")
        pltpu.sync_copy(z_hbm.at[pl.ds(s * DOFF, DSZ)], acc_sh.at[pl.ds(s * DOFF, DSZ)])
        plsc.subcore_barrier()
        base0 = c * epc + s * epsub

        def start_load(base, idx_v, rows_v, sem_i, sem_r):
            pltpu.async_copy(si_hbm.at[pl.ds(base, sw)], idx_v, sem_i)
            pltpu.async_copy(msg_hbm.at[pl.ds(base, sw)], rows_v, sem_r)

        def wait_load(idx_v, rows_v, sem_i, sem_r):
            pltpu.make_async_copy(si_hbm.at[pl.ds(0, sw)], idx_v, sem_i).wait()
            pltpu.make_async_copy(msg_hbm.at[pl.ds(0, sw)], rows_v, sem_r).wait()

        start_load(base0, idx0, rows0, sem_i0, sem_r0)

        @pl.loop(0, nwin, step=2)
        def _(w):
            wait_load(idx0, rows0, sem_i0, sem_r0)
            start_load(base0 + (w + 1) * sw, idx1, rows1, sem_i1, sem_r1)
            a0 = pltpu.async_copy(rows0, acc_sh.at[idx0], sem_a0, add=True)
            wait_load(idx1, rows1, sem_i1, sem_r1)
            a1 = pltpu.async_copy(rows1, acc_sh.at[idx1], sem_a1, add=True)
            a0.wait()

            @pl.when(w + 2 < nwin)
            def _():
                start_load(base0 + (w + 2) * sw, idx0, rows0, sem_i0, sem_r0)

            a1.wait()

        plsc.subcore_barrier()
        pltpu.sync_copy(
            acc_sh.at[pl.ds(s * DOFF, DSZ)],
            o_hbm.at[c, pl.ds(s * DOFF, DSZ)],
        )

    return kern(msg, si, zeros)


def kernel(atom_fea, nbr_fea, self_fea_idx, nbr_fea_idx, params):
    si = self_fea_idx.astype(jnp.int32)
    ni = nbr_fea_idx.astype(jnp.int32)
    si_a, si_b = si[:EH], si[EH:]
    pad = jnp.zeros((EH_PAD - EH,), jnp.int32)
    si_a_pad = jnp.concatenate([si_a, pad])
    si_b_pad = jnp.concatenate([si_b, pad])
    idx_a = jnp.concatenate([si_a, ni[:EH]]).reshape(1, 2 * EH)
    idx_b = jnp.concatenate([si_b, ni[EH:]]).reshape(1, 2 * EH)
    nbr_bf = nbr_fea.astype(jnp.bfloat16)
    nbr_a = nbr_bf[:EH]
    nbr_b = nbr_bf[EH:]
    zeros = jnp.zeros((N, FEA), jnp.float32)
    atom_pad = jnp.pad(atom_fea, ((0, 0), (0, 128 - ELEM_EMB)))
    w_pad = jnp.pad(params["W_emb"], ((0, 128 - ELEM_EMB), (0, 0)))

    x = tc_embed(atom_pad, w_pad, params["b_emb"].reshape(1, FEA))
    for p in params["convs"]:
        W_bf = p["W"].astype(jnp.bfloat16)
        b2d = p["b"].reshape(1, 2 * FEA)
        g1 = p["g1"].reshape(1, 2 * FEA)
        be1 = p["be1"].reshape(1, 2 * FEA)
        G_a = sc_gather(x, idx_a)
        G_b = sc_gather(x, idx_b)
        st_a = tc_stats(G_a, nbr_a, W_bf, b2d)
        st_b = tc_stats(G_b, nbr_b, W_bf, b2d)
        msg_a = tc_msg(G_a, nbr_a, W_bf, b2d, g1, be1, st_a, st_b)
        parts_a = sc_scatter(msg_a, si_a_pad, zeros)
        msg_b = tc_msg(G_b, nbr_b, W_bf, b2d, g1, be1, st_a, st_b)
        parts_b = sc_scatter(msg_b, si_b_pad, zeros)
        x = tc_fapply(x, parts_a, parts_b, p["g2"].reshape(1, FEA), p["be2"].reshape(1, FEA))
    return x


# speedup vs baseline: 1.0014x; 1.0014x over previous
import functools

import jax
import jax.numpy as jnp
from jax import lax
from jax.experimental import pallas as pl
from jax.experimental.pallas import tpu as pltpu
from jax.experimental.pallas import tpu_sc as plsc

N = 10000
E = 320000
ELEM_EMB = 92
NBR = 16
FEA = 128

EH = E // 2
EH_PAD = -(-EH // 4096) * 4096
EB = 1280
NB = 1000
EPS = 1e-5

_NC = 2
_NS = 16
GW = 128
DOFF = 624
DSZ = 640


@functools.cache
def _sc_mesh():
    return plsc.VectorSubcoreMesh(core_axis_name="core", subcore_axis_name="subcore")


def _embed_body(a_ref, w_ref, b_ref, o_ref):
    o_ref[...] = (
        jnp.dot(a_ref[...], w_ref[...], preferred_element_type=jnp.float32)
        + b_ref[...]
    )


def tc_embed(atom_pad, w_pad, b):
    return pl.pallas_call(
        _embed_body,
        grid=(N // NB,),
        in_specs=[
            pl.BlockSpec((NB, 128), lambda i: (i, 0)),
            pl.BlockSpec((128, FEA), lambda i: (0, 0)),
            pl.BlockSpec((1, FEA), lambda i: (0, 0)),
        ],
        out_specs=pl.BlockSpec((NB, FEA), lambda i: (i, 0)),
        out_shape=jax.ShapeDtypeStruct((N, FEA), jnp.float32),
    )(atom_pad, w_pad, b)


def _edge_linear(xs_ref, xn_ref, nb_ref, w_ref, b_ref):
    xs = xs_ref[...].astype(jnp.bfloat16)
    xn = xn_ref[...].astype(jnp.bfloat16)
    nb = nb_ref[...]
    t = jnp.dot(xs, w_ref[0:FEA, :], preferred_element_type=jnp.float32)
    t += jnp.dot(xn, w_ref[FEA:2 * FEA, :], preferred_element_type=jnp.float32)
    t += jnp.dot(nb, w_ref[2 * FEA:2 * FEA + NBR, :], preferred_element_type=jnp.float32)
    return t + b_ref[...]


def _stats_body(xs_ref, xn_ref, nb_ref, w_ref, b_ref, o_ref):
    i = pl.program_id(0)
    t = _edge_linear(xs_ref, xn_ref, nb_ref, w_ref, b_ref)

    @pl.when(i == 0)
    def _():
        o_ref[...] = jnp.zeros_like(o_ref)

    o_ref[0:1, :] += jnp.sum(t, axis=0, keepdims=True)
    o_ref[1:2, :] += jnp.sum(t * t, axis=0, keepdims=True)


def tc_stats(G, nbr_fea, W, b2d):
    neb = G.shape[0] // 2 // EB
    return pl.pallas_call(
        _stats_body,
        grid=(neb,),
        in_specs=[
            pl.BlockSpec((EB, FEA), lambda i: (i, 0)),
            pl.BlockSpec((EB, FEA), lambda i: (neb + i, 0)),
            pl.BlockSpec((EB, NBR), lambda i: (i, 0)),
            pl.BlockSpec((2 * FEA + NBR, 2 * FEA), lambda i: (0, 0)),
            pl.BlockSpec((1, 2 * FEA), lambda i: (0, 0)),
        ],
        out_specs=pl.BlockSpec((8, 2 * FEA), lambda i: (0, 0)),
        out_shape=jax.ShapeDtypeStruct((8, 2 * FEA), jnp.float32),
    )(G, G, nbr_fea, W, b2d)


def _msg_body(xs_ref, xn_ref, nb_ref, w_ref, b_ref, g1_ref, be1_ref,
              sta_ref, stb_ref, o_ref, *, neb_real):
    i = pl.program_id(0)

    @pl.when(i < neb_real)
    def _():
        t = _edge_linear(xs_ref, xn_ref, nb_ref, w_ref, b_ref)
        ssum = sta_ref[0:1, :] + stb_ref[0:1, :]
        ssq = sta_ref[1:2, :] + stb_ref[1:2, :]
        mean = ssum * (1.0 / E)
        var = ssq * (1.0 / E) - mean * mean
        scale = g1_ref[...] * lax.rsqrt(var + EPS)
        y = (t - mean) * scale + be1_ref[...]
        f = y[:, :FEA]
        c = y[:, FEA:]
        sig = 1.0 / (1.0 + jnp.exp(-f))
        sp = jnp.maximum(c, 0.0) + jnp.log1p(jnp.exp(-jnp.abs(c)))
        o_ref[...] = sig * sp

    @pl.when(i >= neb_real)
    def _():
        o_ref[...] = jnp.zeros_like(o_ref)


def tc_msg(G, nbr_fea, W, b2d, g1, be1, sta, stb):
    neb = G.shape[0] // 2 // EB
    nebp = EH_PAD // EB
    clamp = lambda i: jnp.minimum(i, neb - 1)
    return pl.pallas_call(
        functools.partial(_msg_body, neb_real=neb),
        grid=(nebp,),
        in_specs=[
            pl.BlockSpec((EB, FEA), lambda i: (clamp(i), 0)),
            pl.BlockSpec((EB, FEA), lambda i: (neb + clamp(i), 0)),
            pl.BlockSpec((EB, NBR), lambda i: (clamp(i), 0)),
            pl.BlockSpec((2 * FEA + NBR, 2 * FEA), lambda i: (0, 0)),
            pl.BlockSpec((1, 2 * FEA), lambda i: (0, 0)),
            pl.BlockSpec((1, 2 * FEA), lambda i: (0, 0)),
            pl.BlockSpec((1, 2 * FEA), lambda i: (0, 0)),
            pl.BlockSpec((8, 2 * FEA), lambda i: (0, 0)),
            pl.BlockSpec((8, 2 * FEA), lambda i: (0, 0)),
        ],
        out_specs=pl.BlockSpec((EB, FEA), lambda i: (i, 0)),
        out_shape=jax.ShapeDtypeStruct((EH_PAD, FEA), jnp.float32),
    )(G, G, nbr_fea, W, b2d, g1, be1, sta, stb)


def _fapply_body(x_ref, pa0_ref, pa1_ref, pb0_ref, pb1_ref, g2_ref, be2_ref,
                 o_ref, st_ref):
    ph = pl.program_id(0)
    i = pl.program_id(1)
    s = (pa0_ref[0] + pa1_ref[0]) + (pb0_ref[0] + pb1_ref[0])

    @pl.when(ph == 0)
    def _():
        @pl.when(i == 0)
        def _():
            st_ref[...] = jnp.zeros_like(st_ref)

        st_ref[0:1, :] += jnp.sum(s, axis=0, keepdims=True)
        st_ref[1:2, :] += jnp.sum(s * s, axis=0, keepdims=True)

    @pl.when(ph == 1)
    def _():
        mean = st_ref[0:1, :] * (1.0 / N)
        var = st_ref[1:2, :] * (1.0 / N) - mean * mean
        scale = g2_ref[...] * lax.rsqrt(var + EPS)
        y = (s - mean) * scale + be2_ref[...]
        z = x_ref[...] + y
        o_ref[...] = jnp.maximum(z, 0.0) + jnp.log1p(jnp.exp(-jnp.abs(z)))


def tc_fapply(x, parts_a, parts_b, g2, be2):
    return pl.pallas_call(
        _fapply_body,
        grid=(2, N // NB),
        in_specs=[
            pl.BlockSpec((NB, FEA), lambda p, i: (i, 0)),
            pl.BlockSpec((1, NB, FEA), lambda p, i: (0, i, 0)),
            pl.BlockSpec((1, NB, FEA), lambda p, i: (1, i, 0)),
            pl.BlockSpec((1, NB, FEA), lambda p, i: (0, i, 0)),
            pl.BlockSpec((1, NB, FEA), lambda p, i: (1, i, 0)),
            pl.BlockSpec((1, FEA), lambda p, i: (0, 0)),
            pl.BlockSpec((1, FEA), lambda p, i: (0, 0)),
        ],
        out_specs=pl.BlockSpec((NB, FEA), lambda p, i: (i, 0)),
        out_shape=jax.ShapeDtypeStruct((N, FEA), jnp.float32),
        scratch_shapes=[pltpu.VMEM((8, FEA), jnp.float32)],
    )(x, parts_a, parts_a, parts_b, parts_b, g2, be2)


def sc_gather(x, idx2d):
    ng = idx2d.shape[1]

    @functools.partial(
        pl.kernel,
        out_type=jax.ShapeDtypeStruct((ng, FEA), jnp.float32),
        mesh=_sc_mesh(),
        scratch_types=[pltpu.VMEM_SHARED((N, FEA), jnp.float32)],
    )
    def kern(x_hbm, i_hbm, o_hbm, x_sh):
        s = lax.axis_index("subcore")
        pltpu.sync_copy(x_hbm.at[pl.ds(s * DOFF, DSZ)], x_sh.at[pl.ds(s * DOFF, DSZ)])
        plsc.subcore_barrier()

        def body(i_vmem, o_vmem):
            pltpu.sync_copy(x_sh.at[i_vmem.at[0]], o_vmem)

        pltpu.emit_pipeline(
            body,
            grid=(ng // GW,),
            in_specs=[pl.BlockSpec((1, GW), lambda i: (0, i))],
            out_specs=[pl.BlockSpec((GW, FEA), lambda i: (i, 0))],
            core_axis_name=("core", "subcore"),
            dimension_semantics=(pltpu.PARALLEL,),
        )(i_hbm, o_hbm)

    return kern(x, idx2d)


def sc_scatter(msg, si, zeros):
    e_cnt = msg.shape[0]
    epc = e_cnt // _NC
    epsub = epc // _NS
    sw = 128
    nwin = epsub // sw

    @functools.partial(
        pl.kernel,
        out_type=jax.ShapeDtypeStruct((_NC, N, FEA), jnp.float32),
        mesh=_sc_mesh(),
        scratch_types=[
            pltpu.VMEM((sw, FEA), jnp.float32),
            pltpu.VMEM((sw, FEA), jnp.float32),
            pltpu.VMEM((sw,), jnp.int32),
            pltpu.VMEM((sw,), jnp.int32),
            pltpu.VMEM_SHARED((N, FEA), jnp.float32),
            pltpu.SemaphoreType.DMA,
            pltpu.SemaphoreType.DMA,
            pltpu.SemaphoreType.DMA,
            pltpu.SemaphoreType.DMA,
            pltpu.SemaphoreType.DMA,
        ],
    )
    def kern(msg_hbm, si_hbm, z_hbm, o_hbm, rows0, rows1, idx0, idx1, acc_sh,
             sem_i0, sem_r0, sem_i1, sem_r1, sem_a):
        c = lax.axis_index("core")
        s = lax.axis_index("subcore")
        pltpu.sync_copy(z_hbm.at[pl.ds(s * DOFF, DSZ)], acc_sh.at[pl.ds(s * DOFF, DSZ)])
        plsc.subcore_barrier()
        base0 = c * epc + s * epsub

        def start_load(base, idx_v, rows_v, sem_i, sem_r):
            pltpu.async_copy(si_hbm.at[pl.ds(base, sw)], idx_v, sem_i)
            pltpu.async_copy(msg_hbm.at[pl.ds(base, sw)], rows_v, sem_r)

        def wait_load(idx_v, rows_v, sem_i, sem_r):
            pltpu.make_async_copy(si_hbm.at[pl.ds(0, sw)], idx_v, sem_i).wait()
            pltpu.make_async_copy(msg_hbm.at[pl.ds(0, sw)], rows_v, sem_r).wait()

        start_load(base0, idx0, rows0, sem_i0, sem_r0)

        @pl.loop(0, nwin, step=2)
        def _(w):
            wait_load(idx0, rows0, sem_i0, sem_r0)
            start_load(base0 + (w + 1) * sw, idx1, rows1, sem_i1, sem_r1)
            a0 = pltpu.async_copy(rows0, acc_sh.at[idx0], sem_a, add=True)
            wait_load(idx1, rows1, sem_i1, sem_r1)
            a0.wait()

            @pl.when(w + 2 < nwin)
            def _():
                start_load(base0 + (w + 2) * sw, idx0, rows0, sem_i0, sem_r0)

            a1 = pltpu.async_copy(rows1, acc_sh.at[idx1], sem_a, add=True)
            a1.wait()

        plsc.subcore_barrier()
        pltpu.sync_copy(
            acc_sh.at[pl.ds(s * DOFF, DSZ)],
            o_hbm.at[c, pl.ds(s * DOFF, DSZ)],
        )

    return kern(msg, si, zeros)


def kernel(atom_fea, nbr_fea, self_fea_idx, nbr_fea_idx, params):
    si = self_fea_idx.astype(jnp.int32)
    ni = nbr_fea_idx.astype(jnp.int32)
    si_a, si_b = si[:EH], si[EH:]
    pad = jnp.zeros((EH_PAD - EH,), jnp.int32)
    si_a_pad = jnp.concatenate([si_a, pad])
    si_b_pad = jnp.concatenate([si_b, pad])
    idx_a = jnp.concatenate([si_a, ni[:EH]]).reshape(1, 2 * EH)
    idx_b = jnp.concatenate([si_b, ni[EH:]]).reshape(1, 2 * EH)
    nbr_bf = nbr_fea.astype(jnp.bfloat16)
    nbr_a = nbr_bf[:EH]
    nbr_b = nbr_bf[EH:]
    zeros = jnp.zeros((N, FEA), jnp.float32)
    atom_pad = jnp.pad(atom_fea, ((0, 0), (0, 128 - ELEM_EMB)))
    w_pad = jnp.pad(params["W_emb"], ((0, 128 - ELEM_EMB), (0, 0)))

    x = tc_embed(atom_pad, w_pad, params["b_emb"].reshape(1, FEA))
    for p in params["convs"]:
        W_bf = p["W"].astype(jnp.bfloat16)
        b2d = p["b"].reshape(1, 2 * FEA)
        g1 = p["g1"].reshape(1, 2 * FEA)
        be1 = p["be1"].reshape(1, 2 * FEA)
        G_a = sc_gather(x, idx_a)
        G_b = sc_gather(x, idx_b)
        st_a = tc_stats(G_a, nbr_a, W_bf, b2d)
        st_b = tc_stats(G_b, nbr_b, W_bf, b2d)
        msg_a = tc_msg(G_a, nbr_a, W_bf, b2d, g1, be1, st_a, st_b)
        parts_a = sc_scatter(msg_a, si_a_pad, zeros)
        msg_b = tc_msg(G_b, nbr_b, W_bf, b2d, g1, be1, st_a, st_b)
        parts_b = sc_scatter(msg_b, si_b_pad, zeros)
        x = tc_fapply(x, parts_a, parts_b, p["g2"].reshape(1, FEA), p["be2"].reshape(1, FEA))
    return x
